# baseline (speedup 1.0000x reference)
import jax
import jax.numpy as jnp
from jax.experimental import pallas as pl
from jax.experimental.pallas import tpu as pltpu


def _leaky(x):
    return jnp.where(x >= 0, x, 0.01 * x)


def _gcn(x, src, dst, W, b, n):
    xw = x @ W
    loop = jnp.arange(n, dtype=src.dtype)
    s = jnp.concatenate([src, loop])
    d = jnp.concatenate([dst, loop])
    deg = jnp.zeros((n,), xw.dtype).at[d].add(1.0)
    dinv = jnp.where(deg > 0, deg ** -0.5, 0.0)
    norm = (dinv[s] * dinv[d])[:, None]
    out = jnp.zeros_like(xw).at[d].add(norm * xw[s])
    return out + b


def _mlp_h_kernel(flat_ref, w1t_ref, h_ref):
    i = pl.program_id(0)

    @pl.when(i == 0)
    def _():
        h_ref[...] = jnp.zeros_like(h_ref)

    h_ref[...] += flat_ref[...] @ w1t_ref[...]


def _mlp_out_kernel(h_ref, w2t_ref, b2_ref, o_ref):
    o_ref[...] = jnp.maximum(h_ref[...] @ w2t_ref[...] + b2_ref[...], 0.0)


def kernel(x, edge_index, poi_emb, cat_emb, W_in, b_in, Wg0, bg0, Wg1, bg1, Wg2, bg2, Wg3, bg3, Wg4, bg4, W_out, b_out, Wf1, bf1, Wf2, bf2):
    n = x.shape[0]
    poi_idx = x[:, 0].astype(jnp.int32)
    cat_idx = x[:, 1].astype(jnp.int32)
    feat = jnp.concatenate([poi_emb[poi_idx], cat_emb[cat_idx], x[:, 2:5]], axis=1)
    src, dst = edge_index[0], edge_index[1]
    feat = _leaky(_gcn(feat, src, dst, W_in, b_in, n))
    for W, b in ((Wg0, bg0), (Wg1, bg1), (Wg2, bg2), (Wg3, bg3), (Wg4, bg4)):
        t = _gcn(feat, src, dst, W, b, n)
        feat = _leaky(t) + t
    feat = _leaky(_gcn(feat, src, dst, W_out, b_out, n))
    flat = feat.reshape(1, -1)

    NPAD = 38400
    flat_p = jnp.pad(flat, ((0, 0), (0, NPAD - n)))
    w1t = jnp.pad(Wf1.T, ((0, NPAD - Wf1.shape[1]), (0, 0)))
    BN = 2560
    h = pl.pallas_call(
        _mlp_h_kernel,
        grid=(NPAD // BN,),
        in_specs=[
            pl.BlockSpec((1, BN), lambda i: (0, i)),
            pl.BlockSpec((BN, 128), lambda i: (i, 0)),
        ],
        out_specs=pl.BlockSpec((1, 128), lambda i: (0, 0)),
        out_shape=jax.ShapeDtypeStruct((1, 128), jnp.float32),
    )(flat_p, w1t)
    h = jnp.maximum(h + bf1[None, :], 0.0)

    OPAD = 38400
    w2t = jnp.pad(Wf2.T, ((0, 0), (0, OPAD - Wf2.shape[0])))
    b2p = jnp.pad(bf2, (0, OPAD - bf2.shape[0]))[None, :]
    BO = 3840
    out = pl.pallas_call(
        _mlp_out_kernel,
        grid=(OPAD // BO,),
        in_specs=[
            pl.BlockSpec((1, 128), lambda i: (0, 0)),
            pl.BlockSpec((128, BO), lambda i: (0, i)),
            pl.BlockSpec((1, BO), lambda i: (0, i)),
        ],
        out_specs=pl.BlockSpec((1, BO), lambda i: (0, i)),
        out_shape=jax.ShapeDtypeStruct((1, OPAD), jnp.float32),
    )(h, w2t, b2p)
    return out[0, :Wf2.shape[0]]
